# baseline (speedup 1.0000x reference)
import functools

import jax
import jax.numpy as jnp
from jax import lax
from jax.experimental import pallas as pl
from jax.experimental.pallas import tpu as pltpu
from jax.experimental.pallas import tpu_sc as plsc

_V = 1000000
_D = 64
_HD = 32
_NB = 32
_LOGW = 15
_CAP = 8192
_PW = 2048
_NPIECE = (1 << _LOGW) // _PW
_NR = 4


@functools.lru_cache(maxsize=None)
def _build(B: int):
    mesh = plsc.VectorSubcoreMesh(core_axis_name="c", subcore_axis_name="s")
    out_sds = jax.ShapeDtypeStruct((B + 16, 128), jnp.float32)

    @functools.partial(
        pl.kernel,
        mesh=mesh,
        out_type=(out_sds, out_sds),
        scratch_types=[
            pltpu.VMEM((_CAP,), jnp.int32),
            pltpu.VMEM((_CAP,), jnp.int32),
            pltpu.VMEM((_CAP + 16,), jnp.int32),
            pltpu.VMEM((_CAP + 16,), jnp.int32),
            pltpu.VMEM((_HD, _PW), jnp.float32),
            pltpu.VMEM((_NR, 16, 128), jnp.float32),
            pltpu.VMEM((_NB,), jnp.int32),
            pltpu.SemaphoreType.DMA,
        ]
        + [pltpu.SemaphoreType.DMA] * _NR,
        compiler_params=pltpu.CompilerParams(
            use_tc_tiling_on_sc=True, needs_layout_passes=False
        ),
    )
    def k(ridx_hbm, pos_hbm, ct_hbm, tabt_hbm, out0_hbm, out1_hbm,
          rbuf, pbuf, mcol, mpos, strip, ring, ct_v, ssem, *wsems):
        c = lax.axis_index("c")
        s = lax.axis_index("s")
        pltpu.async_copy(ct_hbm, ct_v, ssem).wait()
        iota = lax.iota(jnp.int32, 16)
        ct_lo = ct_v[pl.ds(0, 16)]
        ct_hi = ct_v[pl.ds(16, 16)]

        def bucket_body(cb, carry0):
            w = 2 * s + cb
            lane = w - 16 * lax.div(w, 16)
            in_lo = jnp.full((16,), w < 16)
            n = jnp.sum(
                jnp.where(jnp.logical_and(iota == lane, in_lo), ct_lo, 0)
                + jnp.where(jnp.logical_and(iota == lane, ~in_lo), ct_hi, 0)
            )
            pltpu.sync_copy(ridx_hbm.at[w], rbuf)
            pltpu.sync_copy(pos_hbm.at[w], pbuf)
            n16 = lax.div(n + 15, 16)

            def piece_body(p, carry1):
                p_lo = w * (1 << _LOGW) + p * _PW
                for g in range(4):
                    pltpu.async_copy(
                        tabt_hbm.at[pl.ds((4 * c + g) * 8, 8), pl.ds(p_lo, _PW)],
                        strip.at[pl.ds(8 * g, 8)],
                        ssem,
                    )
                pltpu.make_async_copy(
                    tabt_hbm.at[pl.ds(0, _HD), pl.ds(0, _PW)], strip, ssem
                ).wait()

                def scan_body(i, cur):
                    rv = rbuf[pl.ds(i * 16, 16)]
                    pv = pbuf[pl.ds(i * 16, 16)]
                    m = jnp.logical_and(rv >= p_lo, rv < p_lo + _PW)
                    plsc.store_compressed(mcol.at[pl.ds(cur, 16)], rv - p_lo, mask=m)
                    plsc.store_compressed(mpos.at[pl.ds(cur, 16)], pv, mask=m)
                    return cur + jnp.sum(m.astype(jnp.int32))

                cur = lax.fori_loop(0, n16, scan_body, 0)

                def gather_body(j, carry):
                    colv = mcol[pl.ds(j * 16, 16)]
                    posv = mpos[pl.ds(j * 16, 16)]
                    valid = iota + j * 16 < cur
                    colv = jnp.where(valid, colv, 0)
                    posv = jnp.where(valid, posv, B)
                    b = lax.rem(j, _NR)
                    for rb in range(_NR):

                        @pl.when(jnp.logical_and(b == rb, j >= _NR))
                        def _():
                            @pl.when(c == 0)
                            def _():
                                pltpu.make_async_copy(
                                    out0_hbm.at[pl.ds(0, 16)], ring.at[rb],
                                    wsems[rb],
                                ).wait()

                            @pl.when(c == 1)
                            def _():
                                pltpu.make_async_copy(
                                    out1_hbm.at[pl.ds(0, 16)], ring.at[rb],
                                    wsems[rb],
                                ).wait()

                    bvec = jnp.full((16,), b, jnp.int32)
                    for dd in range(_HD):
                        vals = plsc.load_gather(
                            strip, [jnp.full((16,), dd, jnp.int32), colv]
                        )
                        plsc.store_scatter(
                            ring,
                            [bvec, iota, jnp.full((16,), dd + _HD * c, jnp.int32)],
                            vals,
                        )
                    for rb in range(_NR):

                        @pl.when(b == rb)
                        def _():
                            @pl.when(c == 0)
                            def _():
                                pltpu.async_copy(
                                    ring.at[rb], out0_hbm.at[posv], wsems[rb]
                                )

                            @pl.when(c == 1)
                            def _():
                                pltpu.async_copy(
                                    ring.at[rb], out1_hbm.at[posv], wsems[rb]
                                )
                    return carry

                cur16 = lax.div(cur + 15, 16)
                lax.fori_loop(0, cur16, gather_body, 0)

                pend = jnp.minimum(cur16, _NR)
                for rb in range(_NR):

                    @pl.when(rb < pend)
                    def _():
                        @pl.when(c == 0)
                        def _():
                            pltpu.make_async_copy(
                                out0_hbm.at[pl.ds(0, 16)], ring.at[rb], wsems[rb]
                            ).wait()

                        @pl.when(c == 1)
                        def _():
                            pltpu.make_async_copy(
                                out1_hbm.at[pl.ds(0, 16)], ring.at[rb], wsems[rb]
                            ).wait()
                return carry1

            lax.fori_loop(0, _NPIECE, piece_body, 0)
            return carry0

        lax.fori_loop(0, 2, bucket_body, 0)

    return k


def kernel(inputs, embeddings):
    rows, cols = inputs.shape
    B = rows * cols
    flat = inputs.reshape(B).astype(jnp.int32)
    owner = lax.shift_right_logical(flat, _LOGW)
    oh = (owner[:, None] == jnp.arange(_NB, dtype=jnp.int32)[None, :])
    rank = (
        jnp.take_along_axis(jnp.cumsum(oh.astype(jnp.int32), axis=0),
                            owner[:, None], axis=1)[:, 0] - 1
    )
    dest = owner * _CAP + jnp.minimum(rank, _CAP - 1)
    base_r = jnp.broadcast_to(
        (jnp.arange(_NB, dtype=jnp.int32) << _LOGW)[:, None], (_NB, _CAP)
    ).reshape(-1)
    ridx = base_r.at[dest].set(flat).reshape(_NB, _CAP)
    pos = (
        jnp.full((_NB * _CAP,), B, jnp.int32)
        .at[dest].set(jnp.arange(B, dtype=jnp.int32))
        .reshape(_NB, _CAP)
    )
    counts = jnp.minimum(jnp.sum(oh, axis=0).astype(jnp.int32), _CAP)
    out0, out1 = _build(B)(ridx, pos, counts, embeddings.T)
    out = jnp.concatenate([out0[:B, :_HD], out1[:B, _HD:_D]], axis=1)
    return out.reshape(rows, cols, _D)
